# baseline (speedup 1.0000x reference)
import functools

import jax
import jax.numpy as jnp
from jax import lax
from jax.experimental import pallas as pl
from jax.experimental.pallas import tpu as pltpu
from jax.experimental.pallas import tpu_sc as plsc

_B, _N, _D = 4, 4096, 3
_K = 16
_TQ = 256

_NW = 32
_BN = _B * _N
_QPW = _BN // _NW
_WPB = _N // _QPW


def _knn_body(ptst_ref, q_ref, idx_ref):
    q = q_ref[0]
    p0 = ptst_ref[0, 0:1, :]
    p1 = ptst_ref[0, 1:2, :]
    p2 = ptst_ref[0, 2:3, :]
    d0 = q[:, 0:1] - p0
    d1 = q[:, 1:2] - p1
    d2 = q[:, 2:3] - p2
    dist = (d0 * d0 + d1 * d1) + d2 * d2

    lane_ids = jax.lax.broadcasted_iota(jnp.int32, (_TQ, _N), 1)
    lane_f = lane_ids.astype(jnp.float32)
    row_ids = jax.lax.broadcasted_iota(jnp.int32, (_TQ, _N), 0)
    self_ids = row_ids + pl.program_id(1) * _TQ
    dist = jnp.where(lane_ids == self_ids, jnp.inf, dist)

    for t in range(_K):
        m = jnp.min(dist, axis=1, keepdims=True)
        sel = jnp.min(jnp.where(dist == m, lane_f, jnp.float32(_N)),
                      axis=1, keepdims=True)
        idx_ref[0, :, t : t + 1] = sel.astype(jnp.int32)
        if t + 1 < _K:
            dist = jnp.where(lane_f == sel, jnp.inf, dist)


def _gather_body(pts_hbm, idx_hbm, out_hbm, idx_v, pts_v, o_v):
    wid = lax.axis_index("s") * 2 + lax.axis_index("c")
    b = wid // _WPB
    qbase = (wid % _WPB) * _QPW
    pltpu.sync_copy(idx_hbm.at[pl.ds(wid * _QPW * _K, _QPW * _K)], idx_v)
    pltpu.sync_copy(pts_hbm.at[pl.ds(b * _N * _D, _N * _D)], pts_v)

    lane3 = lax.iota(jnp.int32, _K) * _D

    def body(s, carry):
        vidx = idx_v[pl.ds(s * _K, _K)] * _D
        cidx = jnp.full((_K,), (qbase + s) * _D, jnp.int32)
        obase = s * _K * _D
        for d in range(_D):
            nbr = plsc.load_gather(pts_v, [vidx + d])
            ctr = plsc.load_gather(pts_v, [cidx + d])
            plsc.store_scatter(o_v, [lane3 + (obase + d)], nbr - ctr)
        return carry

    lax.fori_loop(0, _QPW, body, 0)

    base = wid * _QPW * _K * _D
    pltpu.sync_copy(o_v, out_hbm.at[pl.ds(base, _QPW * _K * _D)])


@jax.jit
def _knn_call(points):
    pts_t = jnp.transpose(points, (0, 2, 1))
    idx_out = pl.pallas_call(
        _knn_body,
        grid=(_B, _N // _TQ),
        in_specs=[
            pl.BlockSpec((1, _D, _N), lambda b, q: (b, 0, 0)),
            pl.BlockSpec((1, _TQ, _D), lambda b, q: (b, q, 0)),
        ],
        out_specs=pl.BlockSpec((1, _TQ, _K), lambda b, q: (b, q, 0)),
        out_shape=jax.ShapeDtypeStruct((_B, _N, _K), jnp.int32),
        compiler_params=pltpu.CompilerParams(
            dimension_semantics=("parallel", "parallel"),
        ),
    )(pts_t, points)

    gather = functools.partial(
        pl.kernel,
        mesh=plsc.VectorSubcoreMesh(core_axis_name="c", subcore_axis_name="s"),
        out_type=jax.ShapeDtypeStruct((_BN * _K * _D,), jnp.float32),
        compiler_params=pltpu.CompilerParams(needs_layout_passes=False),
        scratch_types=[
            pltpu.VMEM((_QPW * _K,), jnp.int32),
            pltpu.VMEM((_N * _D,), jnp.float32),
            pltpu.VMEM((_QPW * _K * _D,), jnp.float32),
        ],
    )(_gather_body)
    feat_flat = gather(points.reshape(_BN * _D), idx_out.reshape(_BN * _K))
    return idx_out, feat_flat.reshape(_B, _N, _K, _D)


def kernel(points, k):
    idx_out, local_features = _knn_call(points)
    knn_idx = idx_out + (jnp.asarray(k) - _K).astype(idx_out.dtype)
    return local_features, knn_idx


# speedup vs baseline: 1.1767x; 1.1767x over previous
import functools

import jax
import jax.numpy as jnp
from jax import lax
from jax.experimental import pallas as pl
from jax.experimental.pallas import tpu as pltpu
from jax.experimental.pallas import tpu_sc as plsc

_B, _N, _D = 4, 4096, 3
_K = 16
_TQ = 256

_NW = 32
_BN = _B * _N
_QPW = _BN // _NW
_WPB = _N // _QPW


def _knn_body(ptst_ref, q_ref, idx_ref):
    q = q_ref[0]
    p0 = ptst_ref[0, 0:1, :]
    p1 = ptst_ref[0, 1:2, :]
    p2 = ptst_ref[0, 2:3, :]
    d0 = q[:, 0:1] - p0
    d1 = q[:, 1:2] - p1
    d2 = q[:, 2:3] - p2
    dist = (d0 * d0 + d1 * d1) + d2 * d2

    lane_ids = jax.lax.broadcasted_iota(jnp.int32, (_TQ, _N), 1)
    lane_f = lane_ids.astype(jnp.float32)
    row_ids = jax.lax.broadcasted_iota(jnp.int32, (_TQ, _N), 0)
    self_ids = row_ids + pl.program_id(1) * _TQ
    dist = jnp.where(lane_ids == self_ids, jnp.inf, dist)

    for t in range(_K):
        m = jnp.min(dist, axis=1, keepdims=True)
        sel = jnp.min(jnp.where(dist == m, lane_f, jnp.float32(_N)),
                      axis=1, keepdims=True)
        idx_ref[0, :, t : t + 1] = sel.astype(jnp.int32)
        if t + 1 < _K:
            dist = jnp.where(lane_f == sel, jnp.inf, dist)


def _gather_body(ptsd_hbm, idx_hbm, out_hbm, idx_v, p0_v, p1_v, p2_v,
                 o0_v, o1_v, o2_v):
    wid = lax.axis_index("s") * 2 + lax.axis_index("c")
    b = wid // _WPB
    qbase = (wid % _WPB) * _QPW
    pltpu.sync_copy(idx_hbm.at[pl.ds(wid * _QPW * _K, _QPW * _K)], idx_v)
    pltpu.sync_copy(ptsd_hbm.at[pl.ds(0 * _BN + b * _N, _N)], p0_v)
    pltpu.sync_copy(ptsd_hbm.at[pl.ds(1 * _BN + b * _N, _N)], p1_v)
    pltpu.sync_copy(ptsd_hbm.at[pl.ds(2 * _BN + b * _N, _N)], p2_v)

    def body(s, carry):
        vidx = idx_v[pl.ds(s * _K, _K)]
        cidx = jnp.full((_K,), qbase + s, jnp.int32)
        for p_v, o_v in ((p0_v, o0_v), (p1_v, o1_v), (p2_v, o2_v)):
            nbr = plsc.load_gather(p_v, [vidx])
            ctr = plsc.load_gather(p_v, [cidx])
            o_v[pl.ds(s * _K, _K)] = nbr - ctr
        return carry

    lax.fori_loop(0, _QPW, body, 0)

    base = wid * _QPW * _K
    pltpu.sync_copy(o0_v, out_hbm.at[pl.ds(0 * _BN * _K + base, _QPW * _K)])
    pltpu.sync_copy(o1_v, out_hbm.at[pl.ds(1 * _BN * _K + base, _QPW * _K)])
    pltpu.sync_copy(o2_v, out_hbm.at[pl.ds(2 * _BN * _K + base, _QPW * _K)])


@jax.jit
def _knn_call(points):
    pts_t = jnp.transpose(points, (0, 2, 1))
    idx_out = pl.pallas_call(
        _knn_body,
        grid=(_B, _N // _TQ),
        in_specs=[
            pl.BlockSpec((1, _D, _N), lambda b, q: (b, 0, 0)),
            pl.BlockSpec((1, _TQ, _D), lambda b, q: (b, q, 0)),
        ],
        out_specs=pl.BlockSpec((1, _TQ, _K), lambda b, q: (b, q, 0)),
        out_shape=jax.ShapeDtypeStruct((_B, _N, _K), jnp.int32),
        compiler_params=pltpu.CompilerParams(
            dimension_semantics=("parallel", "parallel"),
        ),
    )(pts_t, points)

    gather = functools.partial(
        pl.kernel,
        mesh=plsc.VectorSubcoreMesh(core_axis_name="c", subcore_axis_name="s"),
        out_type=jax.ShapeDtypeStruct((_D * _BN * _K,), jnp.float32),
        compiler_params=pltpu.CompilerParams(needs_layout_passes=False),
        scratch_types=[
            pltpu.VMEM((_QPW * _K,), jnp.int32),
            pltpu.VMEM((_N,), jnp.float32),
            pltpu.VMEM((_N,), jnp.float32),
            pltpu.VMEM((_N,), jnp.float32),
            pltpu.VMEM((_QPW * _K,), jnp.float32),
            pltpu.VMEM((_QPW * _K,), jnp.float32),
            pltpu.VMEM((_QPW * _K,), jnp.float32),
        ],
    )(_gather_body)
    ptsd = jnp.transpose(points, (2, 0, 1)).reshape(_D * _BN)
    feat_flat = gather(ptsd, idx_out.reshape(_BN * _K))
    feats = jnp.transpose(feat_flat.reshape(_D, _B, _N, _K), (1, 2, 3, 0))
    return idx_out, feats


def kernel(points, k):
    idx_out, local_features = _knn_call(points)
    knn_idx = idx_out + (jnp.asarray(k) - _K).astype(idx_out.dtype)
    return local_features, knn_idx


# speedup vs baseline: 1.1769x; 1.0001x over previous
import functools

import jax
import jax.numpy as jnp
from jax import lax
from jax.experimental import pallas as pl
from jax.experimental.pallas import tpu as pltpu
from jax.experimental.pallas import tpu_sc as plsc

_B, _N, _D = 4, 4096, 3
_K = 16
_TQ = 256

_NW = 32
_BN = _B * _N
_QPW = _BN // _NW
_WPB = _N // _QPW


def _knn_body(ptst_ref, q_ref, idx_ref):
    q = q_ref[0]
    p0 = ptst_ref[0, 0:1, :]
    p1 = ptst_ref[0, 1:2, :]
    p2 = ptst_ref[0, 2:3, :]
    d0 = q[:, 0:1] - p0
    d1 = q[:, 1:2] - p1
    d2 = q[:, 2:3] - p2
    dist = (d0 * d0 + d1 * d1) + d2 * d2

    lane_ids = jax.lax.broadcasted_iota(jnp.int32, (_TQ, _N), 1)
    lane_f = lane_ids.astype(jnp.float32)
    row_ids = jax.lax.broadcasted_iota(jnp.int32, (_TQ, _N), 0)
    self_ids = row_ids + pl.program_id(1) * _TQ
    dist = jnp.where(lane_ids == self_ids, jnp.inf, dist)

    for t in range(_K):
        m = jnp.min(dist, axis=1, keepdims=True)
        sel = jnp.min(jnp.where(dist == m, lane_f, jnp.float32(_N)),
                      axis=1, keepdims=True)
        idx_ref[0, :, t : t + 1] = sel.astype(jnp.int32)
        if t + 1 < _K:
            dist = jnp.where(lane_f == sel, jnp.inf, dist)


def _gather_body(ptsd_hbm, idx_hbm, out_hbm, idx_v, p0_v, p1_v, p2_v,
                 o0_v, o1_v, o2_v):
    wid = lax.axis_index("s") * 2 + lax.axis_index("c")
    b = wid // _WPB
    qbase = (wid % _WPB) * _QPW
    pltpu.sync_copy(idx_hbm.at[pl.ds(wid * _QPW * _K, _QPW * _K)], idx_v)
    pltpu.sync_copy(ptsd_hbm.at[pl.ds((b * _D + 0) * _N, _N)], p0_v)
    pltpu.sync_copy(ptsd_hbm.at[pl.ds((b * _D + 1) * _N, _N)], p1_v)
    pltpu.sync_copy(ptsd_hbm.at[pl.ds((b * _D + 2) * _N, _N)], p2_v)

    def body(s, carry):
        vidx = idx_v[pl.ds(s * _K, _K)]
        cidx = jnp.full((_K,), qbase + s, jnp.int32)
        for p_v, o_v in ((p0_v, o0_v), (p1_v, o1_v), (p2_v, o2_v)):
            nbr = plsc.load_gather(p_v, [vidx])
            ctr = plsc.load_gather(p_v, [cidx])
            o_v[pl.ds(s * _K, _K)] = nbr - ctr
        return carry

    lax.fori_loop(0, _QPW, body, 0)

    base = wid * _QPW * _K
    pltpu.sync_copy(o0_v, out_hbm.at[pl.ds(0 * _BN * _K + base, _QPW * _K)])
    pltpu.sync_copy(o1_v, out_hbm.at[pl.ds(1 * _BN * _K + base, _QPW * _K)])
    pltpu.sync_copy(o2_v, out_hbm.at[pl.ds(2 * _BN * _K + base, _QPW * _K)])


@jax.jit
def _knn_call(points):
    pts_t = jnp.transpose(points, (0, 2, 1))
    idx_out = pl.pallas_call(
        _knn_body,
        grid=(_B, _N // _TQ),
        in_specs=[
            pl.BlockSpec((1, _D, _N), lambda b, q: (b, 0, 0)),
            pl.BlockSpec((1, _TQ, _D), lambda b, q: (b, q, 0)),
        ],
        out_specs=pl.BlockSpec((1, _TQ, _K), lambda b, q: (b, q, 0)),
        out_shape=jax.ShapeDtypeStruct((_B, _N, _K), jnp.int32),
        compiler_params=pltpu.CompilerParams(
            dimension_semantics=("parallel", "parallel"),
        ),
    )(pts_t, points)

    gather = functools.partial(
        pl.kernel,
        mesh=plsc.VectorSubcoreMesh(core_axis_name="c", subcore_axis_name="s"),
        out_type=jax.ShapeDtypeStruct((_D * _BN * _K,), jnp.float32),
        compiler_params=pltpu.CompilerParams(needs_layout_passes=False),
        scratch_types=[
            pltpu.VMEM((_QPW * _K,), jnp.int32),
            pltpu.VMEM((_N,), jnp.float32),
            pltpu.VMEM((_N,), jnp.float32),
            pltpu.VMEM((_N,), jnp.float32),
            pltpu.VMEM((_QPW * _K,), jnp.float32),
            pltpu.VMEM((_QPW * _K,), jnp.float32),
            pltpu.VMEM((_QPW * _K,), jnp.float32),
        ],
    )(_gather_body)
    feat_flat = gather(pts_t.reshape(_B * _D * _N), idx_out.reshape(_BN * _K))
    feats = jnp.transpose(feat_flat.reshape(_D, _B, _N, _K), (1, 2, 3, 0))
    return idx_out, feats


def kernel(points, k):
    idx_out, local_features = _knn_call(points)
    knn_idx = idx_out + (jnp.asarray(k) - _K).astype(idx_out.dtype)
    return local_features, knn_idx
